# Initial kernel scaffold; baseline (speedup 1.0000x reference)
import functools

import jax
import jax.numpy as jnp
from jax import lax
from jax.experimental import pallas as pl
from jax.experimental.pallas import tpu as pltpu
from jax.experimental.pallas import tpu_sc as plsc

N = 10000
E = 320000
D = 128
B = 1024

NC = 2
NS = 16
NW = NC * NS
EPT = E // NW
G = 100
NG = EPT // G
BSUB = B // NS
DSUB = 16
NZ = N // NS

_mesh = plsc.VectorSubcoreMesh(core_axis_name="c", subcore_axis_name="s")


@functools.partial(
    pl.kernel,
    out_type=[
        jax.ShapeDtypeStruct((B, D), jnp.float32),
        jax.ShapeDtypeStruct((NC, B, D), jnp.float32),
        jax.ShapeDtypeStruct((NC, B, DSUB), jnp.float32),
    ],
    mesh=_mesh,
    scratch_types=[
        pltpu.VMEM((NG, G), jnp.int32),
        pltpu.VMEM((NG, G), jnp.int32),
        pltpu.VMEM((G, D), jnp.float32),
        pltpu.VMEM((G, DSUB), jnp.float32),
        pltpu.VMEM((BSUB,), jnp.int32),
        pltpu.VMEM((BSUB, D), jnp.float32),
        pltpu.VMEM((BSUB, DSUB), jnp.float32),
        pltpu.VMEM_SHARED((N, D), jnp.float32),
        pltpu.VMEM_SHARED((N, DSUB), jnp.float32),
        pltpu.SemaphoreType.DMA,
    ],
)
def _sc_aggregate(feat_hbm, src2_hbm, dst2_hbm, bm_hbm, zrows_hbm, zdeg_hbm,
                  center_hbm, neigh_hbm, deg_hbm,
                  src_v, dst_v, rows_v, ones_v, bm_v, brows_v, drows_v,
                  agg_s, deg_s, sem):
    c = lax.axis_index("c")
    s = lax.axis_index("s")
    w = c * NS + s

    pltpu.sync_copy(zrows_hbm, agg_s.at[pl.ds(s * NZ, NZ)])
    pltpu.sync_copy(zdeg_hbm, deg_s.at[pl.ds(s * NZ, NZ)])

    def _init_ones(i, carry):
        ones_v[i, :] = jnp.ones((16,), jnp.float32)
        return carry

    lax.fori_loop(0, G, _init_ones, 0)

    pltpu.sync_copy(src2_hbm.at[pl.ds(w * NG, NG)], src_v)
    pltpu.sync_copy(dst2_hbm.at[pl.ds(w * NG, NG)], dst_v)
    plsc.subcore_barrier()

    def _group(g, carry):
        pltpu.async_copy(feat_hbm.at[src_v.at[g]], rows_v, sem).wait()
        pltpu.sync_copy(rows_v, agg_s.at[dst_v.at[g]], add=True)
        pltpu.sync_copy(ones_v, deg_s.at[dst_v.at[g]], add=True)
        return carry

    lax.fori_loop(0, NG, _group, 0)
    plsc.subcore_barrier()

    pltpu.sync_copy(bm_hbm.at[pl.ds(s * BSUB, BSUB)], bm_v)

    @pl.when(c == 0)
    def _center():
        pltpu.async_copy(feat_hbm.at[bm_v], brows_v, sem).wait()
        pltpu.sync_copy(brows_v, center_hbm.at[pl.ds(s * BSUB, BSUB)])

    pltpu.async_copy(agg_s.at[bm_v], brows_v, sem).wait()
    pltpu.sync_copy(brows_v, neigh_hbm.at[c].at[pl.ds(s * BSUB, BSUB)])
    pltpu.async_copy(deg_s.at[bm_v], drows_v, sem).wait()
    pltpu.sync_copy(drows_v, deg_hbm.at[c].at[pl.ds(s * BSUB, BSUB)])


def _tc_combine_body(center_ref, neighp_ref, degp_ref, wl_ref, wc_ref,
                     emb_ref, log_ref):
    center = center_ref[...]
    neigh = neighp_ref[0] + neighp_ref[1]
    deg = (jnp.sum(degp_ref[0], axis=1) + jnp.sum(degp_ref[1], axis=1)) * (1.0 / DSUB)
    neigh = neigh / jnp.clip(deg, 1.0, None)[:, None]
    wc = wc_ref[...]
    h = (jnp.dot(center, wc[:D], preferred_element_type=jnp.float32)
         + jnp.dot(neigh, wc[D:], preferred_element_type=jnp.float32))
    h = jnp.maximum(h, 0.0)
    norm = jnp.sqrt(jnp.sum(h * h, axis=1, keepdims=True))
    emb_ref[...] = h / (norm + 1e-12)
    log_ref[...] = jnp.dot(center, wl_ref[...], preferred_element_type=jnp.float32)


def _tc_combine(center, neighp, degp, wl, wc):
    return pl.pallas_call(
        _tc_combine_body,
        out_shape=[
            jax.ShapeDtypeStruct((B, D), jnp.float32),
            jax.ShapeDtypeStruct((B, 2), jnp.float32),
        ],
    )(center, neighp, degp, wl, wc)


def kernel(features, labels, batch_mask, train_pos_mask, adj_lists, W_label,
           W_combine):
    src2 = adj_lists[0].reshape(NW * NG, G)
    dst2 = adj_lists[1].reshape(NW * NG, G)
    zrows = jnp.zeros((NZ, D), jnp.float32)
    zdeg = jnp.zeros((NZ, DSUB), jnp.float32)
    center, neighp, degp = _sc_aggregate(features, src2, dst2, batch_mask,
                                         zrows, zdeg)
    embeds, logits = _tc_combine(center, neighp, degp, W_label, W_combine)
    return embeds, logits


# speedup vs baseline: 16.2183x; 16.2183x over previous
import functools

import jax
import jax.numpy as jnp
from jax import lax
from jax.experimental import pallas as pl
from jax.experimental.pallas import tpu as pltpu
from jax.experimental.pallas import tpu_sc as plsc

N = 10000
E = 320000
D = 128
B = 1024

NC = 2
NS = 16
NW = NC * NS
EPT = E // NW
G = 128
BSUB = B // NS
DSUB = 16
DUMMY = B
BPAD = 1040
ZR = BPAD // NS
CBUF = EPT + 2 * G

_mesh = plsc.VectorSubcoreMesh(core_axis_name="c", subcore_axis_name="s")


@functools.partial(
    pl.kernel,
    out_type=[
        jax.ShapeDtypeStruct((B, D), jnp.float32),
        jax.ShapeDtypeStruct((NC, B, D), jnp.float32),
        jax.ShapeDtypeStruct((NC, B, DSUB), jnp.float32),
    ],
    mesh=_mesh,
    compiler_params=pltpu.CompilerParams(needs_layout_passes=False),
    scratch_types=[
        pltpu.VMEM((N,), jnp.int32),
        pltpu.VMEM((EPT,), jnp.int32),
        pltpu.VMEM((EPT,), jnp.int32),
        pltpu.VMEM((CBUF,), jnp.int32),
        pltpu.VMEM((CBUF,), jnp.int32),
        pltpu.VMEM((1, G), jnp.int32),
        pltpu.VMEM((G, D), jnp.float32),
        pltpu.VMEM((G, DSUB), jnp.float32),
        pltpu.VMEM((B,), jnp.int32),
        pltpu.VMEM((BSUB,), jnp.int32),
        pltpu.VMEM((BSUB, D), jnp.float32),
        pltpu.VMEM((BSUB, DSUB), jnp.float32),
        pltpu.VMEM_SHARED((BPAD, D), jnp.float32),
        pltpu.VMEM_SHARED((BPAD, DSUB), jnp.float32),
        pltpu.SemaphoreType.DMA,
    ],
)
def _sc_aggregate(feat_hbm, src_hbm, dst_hbm, bm_hbm, zrows_hbm, zdeg_hbm,
                  center_hbm, neigh_hbm, deg_hbm,
                  slot_v, srcf_v, dstf_v, csrc_v, cslot_v, stage_v, rows_v,
                  ones_v, bm_v, rep_v, brows_v, drows_v,
                  agg_s, deg_s, sem):
    c = lax.axis_index("c")
    s = lax.axis_index("s")
    w = c * NS + s
    iota = lax.iota(jnp.int32, 16)

    pltpu.sync_copy(zrows_hbm, agg_s.at[pl.ds(s * ZR, ZR)])
    pltpu.sync_copy(zdeg_hbm, deg_s.at[pl.ds(s * ZR, ZR)])

    def _init_ones(i, carry):
        ones_v[i, :] = jnp.ones((16,), jnp.float32)
        return carry

    lax.fori_loop(0, G, _init_ones, 0)

    pltpu.sync_copy(bm_hbm, bm_v)
    neg1 = jnp.full((16,), -1, jnp.int32)

    def _init_slot(i, carry):
        slot_v[pl.ds(i * 16, 16)] = neg1
        return carry

    lax.fori_loop(0, N // 16, _init_slot, 0)

    def _scatter_bm(i, carry):
        v_node = bm_v[pl.ds(i * 16, 16)]
        plsc.store_scatter(slot_v, [v_node], i * 16 + iota)
        return carry

    lax.fori_loop(0, B // 16, _scatter_bm, 0)

    pltpu.sync_copy(src_hbm.at[pl.ds(w * EPT, EPT)], srcf_v)
    pltpu.sync_copy(dst_hbm.at[pl.ds(w * EPT, EPT)], dstf_v)

    def _filter(i, off):
        v_d = dstf_v[pl.ds(i * 16, 16)]
        v_s = srcf_v[pl.ds(i * 16, 16)]
        v_slot = plsc.load_gather(slot_v, [v_d])
        m = v_slot >= 0
        mi = m.astype(jnp.int32)
        v_pos = off + plsc.cumsum(mi) - 1
        plsc.store_scatter(csrc_v, [v_pos], v_s, mask=m)
        plsc.store_scatter(cslot_v, [v_pos], v_slot, mask=m)
        return off + jnp.sum(mi)

    off = lax.fori_loop(0, EPT // 16, _filter, jnp.int32(0))

    zero16 = jnp.zeros((16,), jnp.int32)
    dummy16 = jnp.full((16,), DUMMY, jnp.int32)
    for j in range(G // 16):
        v_pos = off + j * 16 + iota
        plsc.store_scatter(csrc_v, [v_pos], zero16)
        plsc.store_scatter(cslot_v, [v_pos], dummy16)
    ngroups = (off + (G - 1)) // G

    plsc.subcore_barrier()

    def _group(g, carry):
        for j in range(G // 16):
            stage_v[0, pl.ds(j * 16, 16)] = cslot_v[pl.ds(g * G + j * 16, 16)]
        pltpu.async_copy(
            feat_hbm.at[csrc_v.at[pl.ds(g * G, G)]], rows_v, sem).wait()
        pltpu.sync_copy(rows_v, agg_s.at[stage_v.at[0]], add=True)
        pltpu.sync_copy(ones_v, deg_s.at[stage_v.at[0]], add=True)
        return carry

    lax.fori_loop(0, ngroups, _group, 0)
    plsc.subcore_barrier()

    @pl.when(c == 0)
    def _center():
        pltpu.async_copy(
            feat_hbm.at[bm_v.at[pl.ds(s * BSUB, BSUB)]], brows_v, sem).wait()
        pltpu.sync_copy(brows_v, center_hbm.at[pl.ds(s * BSUB, BSUB)])

    for j in range(BSUB // 16):
        v_node = bm_v[pl.ds(s * BSUB + j * 16, 16)]
        rep_v[pl.ds(j * 16, 16)] = plsc.load_gather(slot_v, [v_node])

    pltpu.async_copy(agg_s.at[rep_v], brows_v, sem).wait()
    pltpu.sync_copy(brows_v, neigh_hbm.at[c].at[pl.ds(s * BSUB, BSUB)])
    pltpu.async_copy(deg_s.at[rep_v], drows_v, sem).wait()
    pltpu.sync_copy(drows_v, deg_hbm.at[c].at[pl.ds(s * BSUB, BSUB)])


def _tc_combine_body(center_ref, neighp_ref, degp_ref, wl_ref, wc_ref,
                     emb_ref, log_ref):
    center = center_ref[...]
    neigh = neighp_ref[0] + neighp_ref[1]
    deg = (jnp.sum(degp_ref[0], axis=1) + jnp.sum(degp_ref[1], axis=1)) * (1.0 / DSUB)
    neigh = neigh / jnp.clip(deg, 1.0, None)[:, None]
    wc = wc_ref[...]
    h = (jnp.dot(center, wc[:D], preferred_element_type=jnp.float32)
         + jnp.dot(neigh, wc[D:], preferred_element_type=jnp.float32))
    h = jnp.maximum(h, 0.0)
    norm = jnp.sqrt(jnp.sum(h * h, axis=1, keepdims=True))
    emb_ref[...] = h / (norm + 1e-12)
    log_ref[...] = jnp.dot(center, wl_ref[...], preferred_element_type=jnp.float32)


def _tc_combine(center, neighp, degp, wl, wc):
    return pl.pallas_call(
        _tc_combine_body,
        out_shape=[
            jax.ShapeDtypeStruct((B, D), jnp.float32),
            jax.ShapeDtypeStruct((B, 2), jnp.float32),
        ],
    )(center, neighp, degp, wl, wc)


def kernel(features, labels, batch_mask, train_pos_mask, adj_lists, W_label,
           W_combine):
    zrows = jnp.zeros((ZR, D), jnp.float32)
    zdeg = jnp.zeros((ZR, DSUB), jnp.float32)
    center, neighp, degp = _sc_aggregate(features, adj_lists[0], adj_lists[1],
                                         batch_mask, zrows, zdeg)
    embeds, logits = _tc_combine(center, neighp, degp, W_label, W_combine)
    return embeds, logits
